# baseline (speedup 1.0000x reference)
import functools

import jax
import jax.numpy as jnp
from jax import lax
from jax.experimental import pallas as pl
from jax.experimental.pallas import tpu as pltpu
from jax.experimental.pallas import tpu_sc as plsc

BATCH = 16384
D = 64
H1 = 128
H2 = 64
NC = 2
NS = 16
NW = NC * NS
BPW = BATCH // NW

BT = 2048


def _sc_gather(user_table, item_table, uid, iid):
    mesh = plsc.VectorSubcoreMesh(core_axis_name="c", subcore_axis_name="s")

    @functools.partial(
        pl.kernel,
        mesh=mesh,
        out_type=[
            jax.ShapeDtypeStruct((BATCH, D), jnp.float32),
            jax.ShapeDtypeStruct((BATCH, D), jnp.float32),
        ],
        scratch_types=[
            pltpu.VMEM((BPW,), jnp.int32),
            pltpu.VMEM((BPW,), jnp.int32),
            pltpu.SemaphoreType.DMA,
        ],
    )
    def k(ut_hbm, it_hbm, uid_hbm, iid_hbm, uo_hbm, io_hbm,
          uid_v, iid_v, sem):
        wid = lax.axis_index("s") * NC + lax.axis_index("c")
        base = wid * BPW
        pltpu.sync_copy(uid_hbm.at[pl.ds(base, BPW)], uid_v)
        pltpu.sync_copy(iid_hbm.at[pl.ds(base, BPW)], iid_v)

        def run_table(tbl_hbm, id_v, out_hbm):
            @pl.loop(0, BPW, step=16)
            def _(n):
                vec = id_v[pl.ds(n, 16)]
                for l in range(16):
                    pltpu.async_copy(tbl_hbm.at[vec[l]],
                                     out_hbm.at[base + n + l], sem)
            pltpu.make_async_copy(
                tbl_hbm.at[pl.ds(0, BPW)],
                out_hbm.at[pl.ds(base, BPW)], sem).wait()

        run_table(ut_hbm, uid_v, uo_hbm)
        run_table(it_hbm, iid_v, io_hbm)

    return k(user_table, item_table, uid, iid)


def _mlp_body(ue_ref, ie_ref, w1u_ref, w1i_ref, b1_ref, w2_ref, b2_ref,
              w3t_ref, b3_ref, out_ref):
    h = jnp.dot(ue_ref[...], w1u_ref[...], preferred_element_type=jnp.float32)
    h = h + jnp.dot(ie_ref[...], w1i_ref[...],
                    preferred_element_type=jnp.float32)
    h = jax.nn.relu(h + b1_ref[...])
    h = jax.nn.relu(jnp.dot(h, w2_ref[...],
                            preferred_element_type=jnp.float32) + b2_ref[...])
    p = jnp.sum(h * w3t_ref[...], axis=1, keepdims=True) + b3_ref[...]
    out_ref[...] = jax.nn.sigmoid(p)


def _tc_mlp(ue, ie, W1, b1, W2, b2, W3, b3):
    w1u = W1[:D]
    w1i = W1[D:]
    b1r = b1.reshape(1, H1)
    b2r = b2.reshape(1, H2)
    w3t = W3.reshape(1, H2)
    b3r = b3.reshape(1, 1)
    rep = lambda i: (0, 0)
    out = pl.pallas_call(
        _mlp_body,
        grid=(BATCH // BT,),
        in_specs=[
            pl.BlockSpec((BT, D), lambda i: (i, 0)),
            pl.BlockSpec((BT, D), lambda i: (i, 0)),
            pl.BlockSpec((D, H1), rep),
            pl.BlockSpec((D, H1), rep),
            pl.BlockSpec((1, H1), rep),
            pl.BlockSpec((H1, H2), rep),
            pl.BlockSpec((1, H2), rep),
            pl.BlockSpec((1, H2), rep),
            pl.BlockSpec((1, 1), rep),
        ],
        out_specs=pl.BlockSpec((BT, 1), lambda i: (i, 0)),
        out_shape=jax.ShapeDtypeStruct((BATCH, 1), jnp.float32),
    )(ue, ie, w1u, w1i, b1r, W2, b2r, w3t, b3r)
    return out.reshape(BATCH)


def kernel(user_ids, item_ids, user_table, item_table, W1, b1, W2, b2, W3, b3):
    uid = user_ids.astype(jnp.int32)
    iid = item_ids.astype(jnp.int32)
    ue, ie = _sc_gather(user_table, item_table, uid, iid)
    return _tc_mlp(ue, ie, W1, b1, W2, b2, W3, b3)


# speedup vs baseline: 1.6767x; 1.6767x over previous
import functools

import jax
import jax.numpy as jnp
from jax import lax
from jax.experimental import pallas as pl
from jax.experimental.pallas import tpu as pltpu
from jax.experimental.pallas import tpu_sc as plsc

NROWS = 1000000
BATCH = 16384
D = 64
DP = 128
H1 = 128
H2 = 64
NC = 2
NS = 16
NW = NC * NS
BPW = BATCH // NW
CHUNK = 128
K = BPW // CHUNK

TL = 4096
TG = (NROWS + TL - 1) // TL

BT = 2048


def _tr_body(x_ref, o_ref):
    xt = x_ref[...].T
    o_ref[...] = jnp.concatenate(
        [xt, jnp.zeros((TL, DP - D), jnp.float32)], axis=1)


def _tc_transpose(tbl_t):
    return pl.pallas_call(
        _tr_body,
        grid=(TG,),
        in_specs=[pl.BlockSpec((D, TL), lambda i: (0, i))],
        out_specs=pl.BlockSpec((TL, DP), lambda i: (i, 0)),
        out_shape=jax.ShapeDtypeStruct((NROWS, DP), jnp.float32),
    )(tbl_t)


def _sc_gather(ut_pad, it_pad, uid3, iid3):
    mesh = plsc.VectorSubcoreMesh(core_axis_name="c", subcore_axis_name="s")

    @functools.partial(
        pl.kernel,
        mesh=mesh,
        out_type=[
            jax.ShapeDtypeStruct((BATCH, DP), jnp.float32),
            jax.ShapeDtypeStruct((BATCH, DP), jnp.float32),
        ],
        scratch_types=[
            pltpu.VMEM((K, CHUNK), jnp.int32),
            pltpu.VMEM((K, CHUNK), jnp.int32),
            pltpu.VMEM((CHUNK, DP), jnp.float32),
            pltpu.VMEM((CHUNK, DP), jnp.float32),
            pltpu.SemaphoreType.DMA,
            pltpu.SemaphoreType.DMA,
            pltpu.SemaphoreType.DMA,
            pltpu.SemaphoreType.DMA,
        ],
    )
    def k(ut_hbm, it_hbm, uid_hbm, iid_hbm, uo_hbm, io_hbm,
          uidx_v, iidx_v, b0, b1, gs0, gs1, ws0, ws1):
        wid = lax.axis_index("s") * NC + lax.axis_index("c")
        base = wid * BPW
        pltpu.sync_copy(uid_hbm.at[wid], uidx_v)
        pltpu.sync_copy(iid_hbm.at[wid], iidx_v)

        def run_table(tbl, idx_v, out):
            def out_at(c):
                return out.at[pl.ds(base + c * CHUNK, CHUNK)]

            pltpu.async_copy(tbl.at[idx_v.at[0]], b0, gs0)
            pltpu.async_copy(tbl.at[idx_v.at[1]], b1, gs1)
            pltpu.make_async_copy(tbl.at[idx_v.at[0]], b0, gs0).wait()
            pltpu.async_copy(b0, out_at(0), ws0)
            pltpu.make_async_copy(tbl.at[idx_v.at[1]], b1, gs1).wait()
            pltpu.async_copy(b1, out_at(1), ws1)
            pltpu.make_async_copy(b0, out_at(0), ws0).wait()
            pltpu.async_copy(tbl.at[idx_v.at[2]], b0, gs0)
            pltpu.make_async_copy(b1, out_at(1), ws1).wait()
            pltpu.async_copy(tbl.at[idx_v.at[3]], b1, gs1)
            pltpu.make_async_copy(tbl.at[idx_v.at[2]], b0, gs0).wait()
            pltpu.async_copy(b0, out_at(2), ws0)
            pltpu.make_async_copy(tbl.at[idx_v.at[3]], b1, gs1).wait()
            pltpu.async_copy(b1, out_at(3), ws1)
            pltpu.make_async_copy(b0, out_at(2), ws0).wait()
            pltpu.make_async_copy(b1, out_at(3), ws1).wait()

        run_table(ut_hbm, uidx_v, uo_hbm)
        run_table(it_hbm, iidx_v, io_hbm)

    return k(ut_pad, it_pad, uid3, iid3)


def _mlp_body(ue_ref, ie_ref, w1u_ref, w1i_ref, b1_ref, w2_ref, b2_ref,
              w3t_ref, b3_ref, out_ref):
    ue = ue_ref[:, :D]
    ie = ie_ref[:, :D]
    h = jnp.dot(ue, w1u_ref[...], preferred_element_type=jnp.float32)
    h = h + jnp.dot(ie, w1i_ref[...], preferred_element_type=jnp.float32)
    h = jax.nn.relu(h + b1_ref[...])
    h = jax.nn.relu(jnp.dot(h, w2_ref[...],
                            preferred_element_type=jnp.float32) + b2_ref[...])
    p = jnp.sum(h * w3t_ref[...], axis=1, keepdims=True) + b3_ref[...]
    out_ref[...] = jax.nn.sigmoid(p)


def _tc_mlp(ue, ie, W1, b1, W2, b2, W3, b3):
    w1u = W1[:D]
    w1i = W1[D:]
    b1r = b1.reshape(1, H1)
    b2r = b2.reshape(1, H2)
    w3t = W3.reshape(1, H2)
    b3r = b3.reshape(1, 1)
    rep = lambda i: (0, 0)
    out = pl.pallas_call(
        _mlp_body,
        grid=(BATCH // BT,),
        in_specs=[
            pl.BlockSpec((BT, DP), lambda i: (i, 0)),
            pl.BlockSpec((BT, DP), lambda i: (i, 0)),
            pl.BlockSpec((D, H1), rep),
            pl.BlockSpec((D, H1), rep),
            pl.BlockSpec((1, H1), rep),
            pl.BlockSpec((H1, H2), rep),
            pl.BlockSpec((1, H2), rep),
            pl.BlockSpec((1, H2), rep),
            pl.BlockSpec((1, 1), rep),
        ],
        out_specs=pl.BlockSpec((BT, 1), lambda i: (i, 0)),
        out_shape=jax.ShapeDtypeStruct((BATCH, 1), jnp.float32),
    )(ue, ie, w1u, w1i, b1r, W2, b2r, w3t, b3r)
    return out.reshape(BATCH)


def kernel(user_ids, item_ids, user_table, item_table, W1, b1, W2, b2, W3, b3):
    uid3 = user_ids.astype(jnp.int32).reshape(NW, K, CHUNK)
    iid3 = item_ids.astype(jnp.int32).reshape(NW, K, CHUNK)
    ut_pad = _tc_transpose(user_table.T)
    it_pad = _tc_transpose(item_table.T)
    ue, ie = _sc_gather(ut_pad, it_pad, uid3, iid3)
    return _tc_mlp(ue, ie, W1, b1, W2, b2, W3, b3)


# speedup vs baseline: 2.0575x; 1.2271x over previous
import functools

import jax
import jax.numpy as jnp
from jax import lax
from jax.experimental import pallas as pl
from jax.experimental.pallas import tpu as pltpu
from jax.experimental.pallas import tpu_sc as plsc

NROWS = 1000000
SQ = 262144
NPACK = SQ
BATCH = 16384
D = 64
D2 = 32
DP = 128
H1 = 128
H2 = 64
NC = 2
NS = 16
NW = NC * NS
BPW = BATCH // NW
CHUNK = 128
K = BPW // CHUNK

TL = 4096
TG = SQ // TL
QB = SQ // TL

BT = 2048


def _pack_q(x):
    xi = lax.bitcast_convert_type(x, jnp.int32)
    r16 = (xi + 0x8000) >> 16
    w0 = (r16[:D2] & 0xFFFF) | (r16[D2:] << 16)
    return w0.T


def _tr_body(x0_ref, x1_ref, x2_ref, x3_ref, o_ref):
    w = jnp.concatenate(
        [_pack_q(x0_ref[...]), _pack_q(x1_ref[...]),
         _pack_q(x2_ref[...]), _pack_q(x3_ref[...])], axis=1)
    o_ref[...] = lax.bitcast_convert_type(w, jnp.float32)


def _tc_transpose(tbl_t):
    nin = NROWS // TL
    qspec = lambda q: pl.BlockSpec(
        (D, TL), lambda i: (0, jnp.minimum(i + QB * q, nin)))
    return pl.pallas_call(
        _tr_body,
        grid=(TG,),
        in_specs=[qspec(0), qspec(1), qspec(2), qspec(3)],
        out_specs=pl.BlockSpec((TL, DP), lambda i: (i, 0)),
        out_shape=jax.ShapeDtypeStruct((NPACK, DP), jnp.float32),
    )(tbl_t, tbl_t, tbl_t, tbl_t)


def _sc_gather(ut_pad, it_pad, uid3, iid3):
    mesh = plsc.VectorSubcoreMesh(core_axis_name="c", subcore_axis_name="s")

    @functools.partial(
        pl.kernel,
        mesh=mesh,
        out_type=[
            jax.ShapeDtypeStruct((BATCH, DP), jnp.float32),
            jax.ShapeDtypeStruct((BATCH, DP), jnp.float32),
        ],
        scratch_types=[
            pltpu.VMEM((K, CHUNK), jnp.int32),
            pltpu.VMEM((K, CHUNK), jnp.int32),
            pltpu.VMEM((CHUNK, DP), jnp.float32),
            pltpu.VMEM((CHUNK, DP), jnp.float32),
            pltpu.SemaphoreType.DMA,
            pltpu.SemaphoreType.DMA,
            pltpu.SemaphoreType.DMA,
            pltpu.SemaphoreType.DMA,
        ],
    )
    def k(ut_hbm, it_hbm, uid_hbm, iid_hbm, uo_hbm, io_hbm,
          uidx_v, iidx_v, b0, b1, gs0, gs1, ws0, ws1):
        wid = lax.axis_index("s") * NC + lax.axis_index("c")
        base = wid * BPW
        pltpu.sync_copy(uid_hbm.at[wid], uidx_v)
        pltpu.sync_copy(iid_hbm.at[wid], iidx_v)

        def run_table(tbl, idx_v, out):
            def out_at(c):
                return out.at[pl.ds(base + c * CHUNK, CHUNK)]

            pltpu.async_copy(tbl.at[idx_v.at[0]], b0, gs0)
            pltpu.async_copy(tbl.at[idx_v.at[1]], b1, gs1)
            pltpu.make_async_copy(tbl.at[idx_v.at[0]], b0, gs0).wait()
            pltpu.async_copy(b0, out_at(0), ws0)
            pltpu.make_async_copy(tbl.at[idx_v.at[1]], b1, gs1).wait()
            pltpu.async_copy(b1, out_at(1), ws1)
            pltpu.make_async_copy(b0, out_at(0), ws0).wait()
            pltpu.async_copy(tbl.at[idx_v.at[2]], b0, gs0)
            pltpu.make_async_copy(b1, out_at(1), ws1).wait()
            pltpu.async_copy(tbl.at[idx_v.at[3]], b1, gs1)
            pltpu.make_async_copy(tbl.at[idx_v.at[2]], b0, gs0).wait()
            pltpu.async_copy(b0, out_at(2), ws0)
            pltpu.make_async_copy(tbl.at[idx_v.at[3]], b1, gs1).wait()
            pltpu.async_copy(b1, out_at(3), ws1)
            pltpu.make_async_copy(b0, out_at(2), ws0).wait()
            pltpu.make_async_copy(b1, out_at(3), ws1).wait()

        run_table(ut_hbm, uidx_v, uo_hbm)
        run_table(it_hbm, iidx_v, io_hbm)

    return k(ut_pad, it_pad, uid3, iid3)


def _unpack(g, q):
    sel = jnp.zeros((BT, D2), jnp.float32)
    for qq in range(4):
        sel = jnp.where(q == qq, g[:, D2 * qq:D2 * qq + D2], sel)
    bits = lax.bitcast_convert_type(sel, jnp.int32)
    lo = lax.bitcast_convert_type(bits << 16, jnp.float32)
    hi = lax.bitcast_convert_type(bits & jnp.int32(-65536), jnp.float32)
    return lo, hi


def _mlp_body(gu_ref, gi_ref, qu_ref, qi_ref, w1_ref, b1_ref, w2_ref, b2_ref,
              w3t_ref, b3_ref, out_ref):
    ulo, uhi = _unpack(gu_ref[...], qu_ref[...])
    ilo, ihi = _unpack(gi_ref[...], qi_ref[...])
    comb = jnp.concatenate([ulo, uhi, ilo, ihi], axis=1)
    h = jnp.dot(comb, w1_ref[...], preferred_element_type=jnp.float32)
    h = jax.nn.relu(h + b1_ref[...])
    h = jax.nn.relu(jnp.dot(h, w2_ref[...],
                            preferred_element_type=jnp.float32) + b2_ref[...])
    p = jnp.sum(h * w3t_ref[...], axis=1, keepdims=True) + b3_ref[...]
    out_ref[...] = jax.nn.sigmoid(p)


def _tc_mlp(ue, ie, qu, qi, W1, b1, W2, b2, W3, b3):
    b1r = b1.reshape(1, H1)
    b2r = b2.reshape(1, H2)
    w3t = W3.reshape(1, H2)
    b3r = b3.reshape(1, 1)
    rep = lambda i: (0, 0)
    out = pl.pallas_call(
        _mlp_body,
        grid=(BATCH // BT,),
        in_specs=[
            pl.BlockSpec((BT, DP), lambda i: (i, 0)),
            pl.BlockSpec((BT, DP), lambda i: (i, 0)),
            pl.BlockSpec((BT, 1), lambda i: (i, 0)),
            pl.BlockSpec((BT, 1), lambda i: (i, 0)),
            pl.BlockSpec((H1, H1), rep),
            pl.BlockSpec((1, H1), rep),
            pl.BlockSpec((H1, H2), rep),
            pl.BlockSpec((1, H2), rep),
            pl.BlockSpec((1, H2), rep),
            pl.BlockSpec((1, 1), rep),
        ],
        out_specs=pl.BlockSpec((BT, 1), lambda i: (i, 0)),
        out_shape=jax.ShapeDtypeStruct((BATCH, 1), jnp.float32),
    )(ue, ie, qu, qi, W1, b1r, W2, b2r, w3t, b3r)
    return out.reshape(BATCH)


def kernel(user_ids, item_ids, user_table, item_table, W1, b1, W2, b2, W3, b3):
    uids = user_ids.astype(jnp.int32)
    iids = item_ids.astype(jnp.int32)
    uid3 = (uids & (SQ - 1)).reshape(NW, K, CHUNK)
    iid3 = (iids & (SQ - 1)).reshape(NW, K, CHUNK)
    qu = (uids >> 18).reshape(BATCH, 1)
    qi = (iids >> 18).reshape(BATCH, 1)
    ut_pad = _tc_transpose(user_table.T)
    it_pad = _tc_transpose(item_table.T)
    ue, ie = _sc_gather(ut_pad, it_pad, uid3, iid3)
    return _tc_mlp(ue, ie, qu, qi, W1, b1, W2, b2, W3, b3)


# speedup vs baseline: 3.5966x; 1.7481x over previous
import functools

import jax
import jax.numpy as jnp
from jax import lax
from jax.experimental import pallas as pl
from jax.experimental.pallas import tpu as pltpu
from jax.experimental.pallas import tpu_sc as plsc

NROWS = 1000000
SQ = 262144
NPACK = SQ
BATCH = 16384
D = 64
D2 = 32
DP = 128
H1 = 128
H2 = 64
NC = 2
NS = 16
NW = NC * NS
BPW = BATCH // NW
CHUNK = 128
K = BPW // CHUNK

TL = 4096
TG = SQ // TL
QB = SQ // TL

BT = 2048


def _pack_q(x):
    xi = lax.bitcast_convert_type(x, jnp.int32)
    r16 = (xi + 0x8000) >> 16
    return (r16[:D2] & 0xFFFF) | (r16[D2:] << 16)


def _tr_body(x0_ref, x1_ref, x2_ref, x3_ref, o_ref):
    w = jnp.concatenate(
        [_pack_q(x0_ref[...]), _pack_q(x1_ref[...]),
         _pack_q(x2_ref[...]), _pack_q(x3_ref[...])], axis=0)
    o_ref[...] = lax.bitcast_convert_type(w.T, jnp.float32)


def _tc_transpose(tbl_t):
    nin = NROWS // TL
    qspec = lambda q: pl.BlockSpec(
        (D, TL), lambda i: (0, jnp.minimum(i + QB * q, nin)))
    return pl.pallas_call(
        _tr_body,
        grid=(TG,),
        in_specs=[qspec(0), qspec(1), qspec(2), qspec(3)],
        out_specs=pl.BlockSpec((TL, DP), lambda i: (i, 0)),
        out_shape=jax.ShapeDtypeStruct((NPACK, DP), jnp.float32),
    )(tbl_t, tbl_t, tbl_t, tbl_t)


def _sc_gather(ut_pad, it_pad, uid3, iid3):
    mesh = plsc.VectorSubcoreMesh(core_axis_name="c", subcore_axis_name="s")

    @functools.partial(
        pl.kernel,
        mesh=mesh,
        out_type=[
            jax.ShapeDtypeStruct((BATCH, DP), jnp.float32),
            jax.ShapeDtypeStruct((BATCH, DP), jnp.float32),
        ],
        scratch_types=[
            pltpu.VMEM((K, CHUNK), jnp.int32),
            pltpu.VMEM((K, CHUNK), jnp.int32),
            pltpu.VMEM((CHUNK, DP), jnp.float32),
            pltpu.VMEM((CHUNK, DP), jnp.float32),
            pltpu.SemaphoreType.DMA,
            pltpu.SemaphoreType.DMA,
            pltpu.SemaphoreType.DMA,
            pltpu.SemaphoreType.DMA,
        ],
    )
    def k(ut_hbm, it_hbm, uid_hbm, iid_hbm, uo_hbm, io_hbm,
          uidx_v, iidx_v, b0, b1, gs0, gs1, ws0, ws1):
        wid = lax.axis_index("s") * NC + lax.axis_index("c")
        base = wid * BPW
        pltpu.sync_copy(uid_hbm.at[wid], uidx_v)
        pltpu.sync_copy(iid_hbm.at[wid], iidx_v)

        def run_table(tbl, idx_v, out):
            def out_at(c):
                return out.at[pl.ds(base + c * CHUNK, CHUNK)]

            pltpu.async_copy(tbl.at[idx_v.at[0]], b0, gs0)
            pltpu.async_copy(tbl.at[idx_v.at[1]], b1, gs1)
            pltpu.make_async_copy(tbl.at[idx_v.at[0]], b0, gs0).wait()
            pltpu.async_copy(b0, out_at(0), ws0)
            pltpu.make_async_copy(tbl.at[idx_v.at[1]], b1, gs1).wait()
            pltpu.async_copy(b1, out_at(1), ws1)
            pltpu.make_async_copy(b0, out_at(0), ws0).wait()
            pltpu.async_copy(tbl.at[idx_v.at[2]], b0, gs0)
            pltpu.make_async_copy(b1, out_at(1), ws1).wait()
            pltpu.async_copy(tbl.at[idx_v.at[3]], b1, gs1)
            pltpu.make_async_copy(tbl.at[idx_v.at[2]], b0, gs0).wait()
            pltpu.async_copy(b0, out_at(2), ws0)
            pltpu.make_async_copy(tbl.at[idx_v.at[3]], b1, gs1).wait()
            pltpu.async_copy(b1, out_at(3), ws1)
            pltpu.make_async_copy(b0, out_at(2), ws0).wait()
            pltpu.make_async_copy(b1, out_at(3), ws1).wait()

        run_table(ut_hbm, uidx_v, uo_hbm)
        run_table(it_hbm, iidx_v, io_hbm)

    return k(ut_pad, it_pad, uid3, iid3)


def _unpack(g, q):
    sel = jnp.zeros((BT, D2), jnp.float32)
    for qq in range(4):
        sel = jnp.where(q == qq, g[:, D2 * qq:D2 * qq + D2], sel)
    bits = lax.bitcast_convert_type(sel, jnp.int32)
    lo = lax.bitcast_convert_type(bits << 16, jnp.float32)
    hi = lax.bitcast_convert_type(bits & jnp.int32(-65536), jnp.float32)
    return lo, hi


def _mlp_body(gu_ref, gi_ref, qu_ref, qi_ref, w1_ref, b1_ref, w2_ref, b2_ref,
              w3t_ref, b3_ref, out_ref):
    ulo, uhi = _unpack(gu_ref[...], qu_ref[...])
    ilo, ihi = _unpack(gi_ref[...], qi_ref[...])
    comb = jnp.concatenate([ulo, uhi, ilo, ihi], axis=1)
    h = jnp.dot(comb, w1_ref[...], preferred_element_type=jnp.float32)
    h = jax.nn.relu(h + b1_ref[...])
    h = jax.nn.relu(jnp.dot(h, w2_ref[...],
                            preferred_element_type=jnp.float32) + b2_ref[...])
    p = jnp.sum(h * w3t_ref[...], axis=1, keepdims=True) + b3_ref[...]
    out_ref[...] = jax.nn.sigmoid(p)


def _tc_mlp(ue, ie, qu, qi, W1, b1, W2, b2, W3, b3):
    b1r = b1.reshape(1, H1)
    b2r = b2.reshape(1, H2)
    w3t = W3.reshape(1, H2)
    b3r = b3.reshape(1, 1)
    rep = lambda i: (0, 0)
    out = pl.pallas_call(
        _mlp_body,
        grid=(BATCH // BT,),
        in_specs=[
            pl.BlockSpec((BT, DP), lambda i: (i, 0)),
            pl.BlockSpec((BT, DP), lambda i: (i, 0)),
            pl.BlockSpec((BT, 1), lambda i: (i, 0)),
            pl.BlockSpec((BT, 1), lambda i: (i, 0)),
            pl.BlockSpec((H1, H1), rep),
            pl.BlockSpec((1, H1), rep),
            pl.BlockSpec((H1, H2), rep),
            pl.BlockSpec((1, H2), rep),
            pl.BlockSpec((1, H2), rep),
            pl.BlockSpec((1, 1), rep),
        ],
        out_specs=pl.BlockSpec((BT, 1), lambda i: (i, 0)),
        out_shape=jax.ShapeDtypeStruct((BATCH, 1), jnp.float32),
    )(ue, ie, qu, qi, W1, b1r, W2, b2r, w3t, b3r)
    return out.reshape(BATCH)


def kernel(user_ids, item_ids, user_table, item_table, W1, b1, W2, b2, W3, b3):
    uids = user_ids.astype(jnp.int32)
    iids = item_ids.astype(jnp.int32)
    uid3 = (uids & (SQ - 1)).reshape(NW, K, CHUNK)
    iid3 = (iids & (SQ - 1)).reshape(NW, K, CHUNK)
    qu = (uids >> 18).reshape(BATCH, 1)
    qi = (iids >> 18).reshape(BATCH, 1)
    ut_pad = _tc_transpose(user_table.T)
    it_pad = _tc_transpose(item_table.T)
    ue, ie = _sc_gather(ut_pad, it_pad, uid3, iid3)
    return _tc_mlp(ue, ie, qu, qi, W1, b1, W2, b2, W3, b3)


# speedup vs baseline: 3.6174x; 1.0058x over previous
import functools

import jax
import jax.numpy as jnp
from jax import lax
from jax.experimental import pallas as pl
from jax.experimental.pallas import tpu as pltpu
from jax.experimental.pallas import tpu_sc as plsc

NROWS = 1000000
SQ = 262144
NPACK = SQ
BATCH = 16384
D = 64
D2 = 32
DP = 128
H1 = 128
H2 = 64
NC = 2
NS = 16
NW = NC * NS
BPW = BATCH // NW
CHUNK = 128
K = BPW // CHUNK

TL = 4096
TG = SQ // TL
QB = SQ // TL

BT = 2048


def _pack_q(x):
    xi = lax.bitcast_convert_type(x, jnp.int32)
    r16 = (xi + 0x8000) >> 16
    return (r16[:D2] & 0xFFFF) | (r16[D2:] << 16)


def _tr_body(x0_ref, x1_ref, x2_ref, x3_ref, o_ref):
    w = jnp.concatenate(
        [_pack_q(x0_ref[...]), _pack_q(x1_ref[...]),
         _pack_q(x2_ref[...]), _pack_q(x3_ref[...])], axis=0)
    o_ref[...] = lax.bitcast_convert_type(w.T, jnp.float32)


def _tc_transpose(tbl_t):
    nin = NROWS // TL
    qspec = lambda q: pl.BlockSpec(
        (D, TL), lambda i: (0, jnp.minimum(i + QB * q, nin)))
    return pl.pallas_call(
        _tr_body,
        grid=(TG,),
        in_specs=[qspec(0), qspec(1), qspec(2), qspec(3)],
        out_specs=pl.BlockSpec((TL, DP), lambda i: (i, 0)),
        out_shape=jax.ShapeDtypeStruct((NPACK, DP), jnp.float32),
    )(tbl_t, tbl_t, tbl_t, tbl_t)


def _sc_gather(tbl_pad, id3):
    mesh = plsc.VectorSubcoreMesh(core_axis_name="c", subcore_axis_name="s")

    @functools.partial(
        pl.kernel,
        mesh=mesh,
        out_type=jax.ShapeDtypeStruct((BATCH, DP), jnp.float32),
        scratch_types=[
            pltpu.VMEM((K, CHUNK), jnp.int32),
            pltpu.VMEM((CHUNK, DP), jnp.float32),
            pltpu.VMEM((CHUNK, DP), jnp.float32),
            pltpu.SemaphoreType.DMA,
            pltpu.SemaphoreType.DMA,
            pltpu.SemaphoreType.DMA,
            pltpu.SemaphoreType.DMA,
        ],
    )
    def k(tbl_hbm, id_hbm, out_hbm, idx_v, b0, b1, gs0, gs1, ws0, ws1):
        wid = lax.axis_index("s") * NC + lax.axis_index("c")
        base = wid * BPW
        pltpu.sync_copy(id_hbm.at[wid], idx_v)

        def out_at(c):
            return out_hbm.at[pl.ds(base + c * CHUNK, CHUNK)]

        pltpu.async_copy(tbl_hbm.at[idx_v.at[0]], b0, gs0)
        pltpu.async_copy(tbl_hbm.at[idx_v.at[1]], b1, gs1)
        pltpu.make_async_copy(tbl_hbm.at[idx_v.at[0]], b0, gs0).wait()
        pltpu.async_copy(b0, out_at(0), ws0)
        pltpu.make_async_copy(tbl_hbm.at[idx_v.at[1]], b1, gs1).wait()
        pltpu.async_copy(b1, out_at(1), ws1)
        pltpu.make_async_copy(b0, out_at(0), ws0).wait()
        pltpu.async_copy(tbl_hbm.at[idx_v.at[2]], b0, gs0)
        pltpu.make_async_copy(b1, out_at(1), ws1).wait()
        pltpu.async_copy(tbl_hbm.at[idx_v.at[3]], b1, gs1)
        pltpu.make_async_copy(tbl_hbm.at[idx_v.at[2]], b0, gs0).wait()
        pltpu.async_copy(b0, out_at(2), ws0)
        pltpu.make_async_copy(tbl_hbm.at[idx_v.at[3]], b1, gs1).wait()
        pltpu.async_copy(b1, out_at(3), ws1)
        pltpu.make_async_copy(b0, out_at(2), ws0).wait()
        pltpu.make_async_copy(b1, out_at(3), ws1).wait()

    return k(tbl_pad, id3)


def _unpack(g, q):
    sel = jnp.zeros((BT, D2), jnp.float32)
    for qq in range(4):
        sel = jnp.where(q == qq, g[:, D2 * qq:D2 * qq + D2], sel)
    bits = lax.bitcast_convert_type(sel, jnp.int32)
    lo = lax.bitcast_convert_type(bits << 16, jnp.float32)
    hi = lax.bitcast_convert_type(bits & jnp.int32(-65536), jnp.float32)
    return lo, hi


def _mlp_body(gu_ref, gi_ref, qu_ref, qi_ref, w1_ref, b1_ref, w2_ref, b2_ref,
              w3t_ref, b3_ref, out_ref):
    ulo, uhi = _unpack(gu_ref[...], qu_ref[...])
    ilo, ihi = _unpack(gi_ref[...], qi_ref[...])
    comb = jnp.concatenate([ulo, uhi, ilo, ihi], axis=1)
    h = jnp.dot(comb, w1_ref[...], preferred_element_type=jnp.float32)
    h = jax.nn.relu(h + b1_ref[...])
    h = jax.nn.relu(jnp.dot(h, w2_ref[...],
                            preferred_element_type=jnp.float32) + b2_ref[...])
    p = jnp.sum(h * w3t_ref[...], axis=1, keepdims=True) + b3_ref[...]
    out_ref[...] = jax.nn.sigmoid(p)


def _tc_mlp(ue, ie, qu, qi, W1, b1, W2, b2, W3, b3):
    b1r = b1.reshape(1, H1)
    b2r = b2.reshape(1, H2)
    w3t = W3.reshape(1, H2)
    b3r = b3.reshape(1, 1)
    rep = lambda i: (0, 0)
    out = pl.pallas_call(
        _mlp_body,
        grid=(BATCH // BT,),
        in_specs=[
            pl.BlockSpec((BT, DP), lambda i: (i, 0)),
            pl.BlockSpec((BT, DP), lambda i: (i, 0)),
            pl.BlockSpec((BT, 1), lambda i: (i, 0)),
            pl.BlockSpec((BT, 1), lambda i: (i, 0)),
            pl.BlockSpec((H1, H1), rep),
            pl.BlockSpec((1, H1), rep),
            pl.BlockSpec((H1, H2), rep),
            pl.BlockSpec((1, H2), rep),
            pl.BlockSpec((1, H2), rep),
            pl.BlockSpec((1, 1), rep),
        ],
        out_specs=pl.BlockSpec((BT, 1), lambda i: (i, 0)),
        out_shape=jax.ShapeDtypeStruct((BATCH, 1), jnp.float32),
    )(ue, ie, qu, qi, W1, b1r, W2, b2r, w3t, b3r)
    return out.reshape(BATCH)


def kernel(user_ids, item_ids, user_table, item_table, W1, b1, W2, b2, W3, b3):
    uids = user_ids.astype(jnp.int32)
    iids = item_ids.astype(jnp.int32)
    uid3 = (uids & (SQ - 1)).reshape(NW, K, CHUNK)
    iid3 = (iids & (SQ - 1)).reshape(NW, K, CHUNK)
    qu = (uids >> 18).reshape(BATCH, 1)
    qi = (iids >> 18).reshape(BATCH, 1)
    ut_pad = _tc_transpose(user_table.T)
    ue = _sc_gather(ut_pad, uid3)
    it_pad = _tc_transpose(item_table.T)
    ie = _sc_gather(it_pad, iid3)
    return _tc_mlp(ue, ie, qu, qi, W1, b1, W2, b2, W3, b3)


# speedup vs baseline: 3.8754x; 1.0713x over previous
import functools

import jax
import jax.numpy as jnp
from jax import lax
from jax.experimental import pallas as pl
from jax.experimental.pallas import tpu as pltpu
from jax.experimental.pallas import tpu_sc as plsc

NROWS = 1000000
SQ = 262144
NPACK = SQ
BATCH = 16384
D = 64
D2 = 32
DP = 128
H1 = 128
H2 = 64
NC = 2
NS = 16
NW = NC * NS
BPW = BATCH // NW
CHUNK = 128
K = BPW // CHUNK

TL = 8192
TG = SQ // TL
QB = SQ // TL

BT = 4096


def _pack_q(x):
    xi = lax.bitcast_convert_type(x, jnp.int32)
    r16 = (xi + 0x8000) >> 16
    return (r16[:D2] & 0xFFFF) | (r16[D2:] << 16)


def _tr_body(x0_ref, x1_ref, x2_ref, x3_ref, o_ref):
    w = jnp.concatenate(
        [_pack_q(x0_ref[...]), _pack_q(x1_ref[...]),
         _pack_q(x2_ref[...]), _pack_q(x3_ref[...])], axis=0)
    o_ref[...] = lax.bitcast_convert_type(w.T, jnp.float32)


def _tc_transpose(tbl_t):
    nin = NROWS // TL
    qspec = lambda q: pl.BlockSpec(
        (D, TL), lambda i: (0, jnp.minimum(i + QB * q, nin)))
    return pl.pallas_call(
        _tr_body,
        grid=(TG,),
        in_specs=[qspec(0), qspec(1), qspec(2), qspec(3)],
        out_specs=pl.BlockSpec((TL, DP), lambda i: (i, 0)),
        out_shape=jax.ShapeDtypeStruct((NPACK, DP), jnp.float32),
    )(tbl_t, tbl_t, tbl_t, tbl_t)


def _sc_gather(tbl_pad, id3):
    mesh = plsc.VectorSubcoreMesh(core_axis_name="c", subcore_axis_name="s")

    @functools.partial(
        pl.kernel,
        mesh=mesh,
        out_type=jax.ShapeDtypeStruct((BATCH, DP), jnp.float32),
        scratch_types=[
            pltpu.VMEM((K, CHUNK), jnp.int32),
            pltpu.VMEM((CHUNK, DP), jnp.float32),
            pltpu.VMEM((CHUNK, DP), jnp.float32),
            pltpu.SemaphoreType.DMA,
            pltpu.SemaphoreType.DMA,
            pltpu.SemaphoreType.DMA,
            pltpu.SemaphoreType.DMA,
        ],
    )
    def k(tbl_hbm, id_hbm, out_hbm, idx_v, b0, b1, gs0, gs1, ws0, ws1):
        wid = lax.axis_index("s") * NC + lax.axis_index("c")
        base = wid * BPW
        pltpu.sync_copy(id_hbm.at[wid], idx_v)

        def out_at(c):
            return out_hbm.at[pl.ds(base + c * CHUNK, CHUNK)]

        pltpu.async_copy(tbl_hbm.at[idx_v.at[0]], b0, gs0)
        pltpu.async_copy(tbl_hbm.at[idx_v.at[1]], b1, gs1)
        pltpu.make_async_copy(tbl_hbm.at[idx_v.at[0]], b0, gs0).wait()
        pltpu.async_copy(b0, out_at(0), ws0)
        pltpu.make_async_copy(tbl_hbm.at[idx_v.at[1]], b1, gs1).wait()
        pltpu.async_copy(b1, out_at(1), ws1)
        pltpu.make_async_copy(b0, out_at(0), ws0).wait()
        pltpu.async_copy(tbl_hbm.at[idx_v.at[2]], b0, gs0)
        pltpu.make_async_copy(b1, out_at(1), ws1).wait()
        pltpu.async_copy(tbl_hbm.at[idx_v.at[3]], b1, gs1)
        pltpu.make_async_copy(tbl_hbm.at[idx_v.at[2]], b0, gs0).wait()
        pltpu.async_copy(b0, out_at(2), ws0)
        pltpu.make_async_copy(tbl_hbm.at[idx_v.at[3]], b1, gs1).wait()
        pltpu.async_copy(b1, out_at(3), ws1)
        pltpu.make_async_copy(b0, out_at(2), ws0).wait()
        pltpu.make_async_copy(b1, out_at(3), ws1).wait()

    return k(tbl_pad, id3)


def _unpack(g, q):
    laneq = lax.broadcasted_iota(jnp.int32, (BT, DP), 1) >> 5
    gm = jnp.where(laneq == q, lax.bitcast_convert_type(g, jnp.int32), 0)
    bits = (gm[:, :D2] | gm[:, D2:2 * D2]
            | gm[:, 2 * D2:3 * D2] | gm[:, 3 * D2:])
    lo = lax.bitcast_convert_type(bits << 16, jnp.float32)
    hi = lax.bitcast_convert_type(bits & jnp.int32(-65536), jnp.float32)
    return lo, hi


def _mlp_body(gu_ref, gi_ref, qu_ref, qi_ref, w1_ref, b1_ref, w2_ref, b2_ref,
              w3t_ref, b3_ref, out_ref):
    ulo, uhi = _unpack(gu_ref[...], qu_ref[...])
    ilo, ihi = _unpack(gi_ref[...], qi_ref[...])
    comb = jnp.concatenate([ulo, uhi, ilo, ihi], axis=1)
    h = jnp.dot(comb, w1_ref[...], preferred_element_type=jnp.float32)
    h = jax.nn.relu(h + b1_ref[...])
    h = jax.nn.relu(jnp.dot(h, w2_ref[...],
                            preferred_element_type=jnp.float32) + b2_ref[...])
    p = jnp.sum(h * w3t_ref[...], axis=1, keepdims=True) + b3_ref[...]
    out_ref[...] = jax.nn.sigmoid(p)


def _tc_mlp(ue, ie, qu, qi, W1, b1, W2, b2, W3, b3):
    b1r = b1.reshape(1, H1)
    b2r = b2.reshape(1, H2)
    w3t = W3.reshape(1, H2)
    b3r = b3.reshape(1, 1)
    rep = lambda i: (0, 0)
    out = pl.pallas_call(
        _mlp_body,
        grid=(BATCH // BT,),
        in_specs=[
            pl.BlockSpec((BT, DP), lambda i: (i, 0)),
            pl.BlockSpec((BT, DP), lambda i: (i, 0)),
            pl.BlockSpec((BT, 1), lambda i: (i, 0)),
            pl.BlockSpec((BT, 1), lambda i: (i, 0)),
            pl.BlockSpec((H1, H1), rep),
            pl.BlockSpec((1, H1), rep),
            pl.BlockSpec((H1, H2), rep),
            pl.BlockSpec((1, H2), rep),
            pl.BlockSpec((1, H2), rep),
            pl.BlockSpec((1, 1), rep),
        ],
        out_specs=pl.BlockSpec((BT, 1), lambda i: (i, 0)),
        out_shape=jax.ShapeDtypeStruct((BATCH, 1), jnp.float32),
    )(ue, ie, qu, qi, W1, b1r, W2, b2r, w3t, b3r)
    return out.reshape(BATCH)


def kernel(user_ids, item_ids, user_table, item_table, W1, b1, W2, b2, W3, b3):
    uids = user_ids.astype(jnp.int32)
    iids = item_ids.astype(jnp.int32)
    uid3 = (uids & (SQ - 1)).reshape(NW, K, CHUNK)
    iid3 = (iids & (SQ - 1)).reshape(NW, K, CHUNK)
    qu = (uids >> 18).reshape(BATCH, 1)
    qi = (iids >> 18).reshape(BATCH, 1)
    ut_pad = _tc_transpose(user_table.T)
    ue = _sc_gather(ut_pad, uid3)
    it_pad = _tc_transpose(item_table.T)
    ie = _sc_gather(it_pad, iid3)
    return _tc_mlp(ue, ie, qu, qi, W1, b1, W2, b2, W3, b3)
